# Initial kernel scaffold; baseline (speedup 1.0000x reference)
import jax
import jax.numpy as jnp
from jax.experimental import pallas as pl


def kernel(x, edge_index, batch, feature_prompt, readout_prompt, W_gnn):
    raise NotImplementedError("write your pallas kernel here")


# speedup vs baseline: 12.5448x; 12.5448x over previous
import jax
import jax.numpy as jnp
from jax import lax
from jax.experimental import pallas as pl
from jax.experimental.pallas import tpu as pltpu
from jax.experimental.pallas import tpu_sc as plsc

N = 10000
NB = 10240
NPC = 10008
E = 160000
G = 64
D = 256

NTILES = 16
EROWS = 80
LANES = 128
EPT = EROWS * LANES
EPAD = NTILES * EPT
BROWS = NB // LANES
CW = G * NPC
CC = CW // NTILES
TMPW = CC // 4


def _sc_body(src_hbm, dst_hbm, batch2_hbm,
             c_hbm, cnt_hbm,
             src_v, dst_v, batch_v, deg_v, alpha_v, fidx_v,
             ones_v, tmp_v, cnt_v, c_sh, deg_sh, cnt_sh):
    wid = lax.axis_index("s")

    pltpu.sync_copy(src_hbm.at[pl.ds(wid * EROWS, EROWS)], src_v)
    pltpu.sync_copy(dst_hbm.at[pl.ds(wid * EROWS, EROWS)], dst_v)
    for j in range(BROWS):
        pltpu.sync_copy(batch2_hbm.at[j], batch_v.at[pl.ds(j * LANES, LANES)])

    def _zero(i, _):
        tmp_v[pl.ds(i * 16, 16)] = jnp.zeros((16,), jnp.float32)
        return 0

    lax.fori_loop(0, (TMPW + 15) // 16, _zero, 0)
    for k in range(4):
        pltpu.sync_copy(tmp_v.at[pl.ds(0, TMPW)],
                        c_sh.at[pl.ds(wid * CC + k * TMPW, TMPW)])

    @pl.when(wid == 0)
    def _():
        pltpu.sync_copy(tmp_v.at[pl.ds(0, TMPW)], deg_sh.at[pl.ds(0, TMPW)])
        pltpu.sync_copy(tmp_v.at[pl.ds(0, NB - TMPW)],
                        deg_sh.at[pl.ds(TMPW, NB - TMPW)])
        pltpu.sync_copy(tmp_v.at[pl.ds(0, 2 * LANES)], cnt_sh)

    for i in range(LANES // 16):
        ones_v[pl.ds(i * 16, 16)] = jnp.ones((16,), jnp.float32)

    plsc.subcore_barrier()

    for j in range(EROWS):
        pltpu.sync_copy(ones_v, deg_sh.at[dst_v.at[j]], add=True)

    @pl.when(wid == 0)
    def _():
        pltpu.sync_copy(batch2_hbm, fidx_v)
        for c in range(1, LANES // 16):
            fidx_v[BROWS - 2, pl.ds(c * 16, 16)] = jnp.full((16,), G,
                                                            jnp.int32)
        for c in range(LANES // 16):
            fidx_v[BROWS - 1, pl.ds(c * 16, 16)] = jnp.full((16,), G,
                                                            jnp.int32)
        for j in range(BROWS):
            pltpu.sync_copy(ones_v, cnt_sh.at[fidx_v.at[j]], add=True)

    plsc.subcore_barrier()

    pltpu.sync_copy(deg_sh, deg_v)

    def _inv(i, _):
        v = deg_v[pl.ds(i * 16, 16)]
        deg_v[pl.ds(i * 16, 16)] = 1.0 / jnp.maximum(v, 1.0)
        return 0

    lax.fori_loop(0, NB // 16, _inv, 0)

    def _edges(r, _):
        for c in range(LANES // 16):
            sl = pl.ds(c * 16, 16)
            d = dst_v[r, sl]
            s = src_v[r, sl]
            alpha_v[r, sl] = plsc.load_gather(deg_v, [d])
            g = plsc.load_gather(batch_v, [d])
            fidx_v[r, sl] = g * NPC + s
        return 0

    lax.fori_loop(0, EROWS, _edges, 0)

    for j in range(EROWS):
        pltpu.sync_copy(alpha_v.at[j], c_sh.at[fidx_v.at[j]], add=True)

    plsc.subcore_barrier()

    for k in range(4):
        off = wid * CC + k * TMPW
        pltpu.sync_copy(c_sh.at[pl.ds(off, TMPW)], tmp_v.at[pl.ds(0, TMPW)])
        pltpu.sync_copy(tmp_v.at[pl.ds(0, TMPW)], c_hbm.at[pl.ds(off, TMPW)])

    @pl.when(wid == 0)
    def _():
        pltpu.sync_copy(cnt_sh.at[pl.ds(0, G)], cnt_v)
        pltpu.sync_copy(cnt_v, cnt_hbm)


def _sc_build_fn(interpret):
    mesh = plsc.VectorSubcoreMesh(core_axis_name="c", subcore_axis_name="s",
                                  num_cores=1, num_subcores=NTILES)
    return pl.kernel(
        _sc_body,
        out_type=(jax.ShapeDtypeStruct((CW,), jnp.float32),
                  jax.ShapeDtypeStruct((G,), jnp.float32)),
        mesh=mesh,
        compiler_params=pltpu.CompilerParams(needs_layout_passes=False),
        interpret=interpret,
        scratch_types=[
            pltpu.VMEM((EROWS, LANES), jnp.int32),
            pltpu.VMEM((EROWS, LANES), jnp.int32),
            pltpu.VMEM((NB,), jnp.int32),
            pltpu.VMEM((NB,), jnp.float32),
            pltpu.VMEM((EROWS, LANES), jnp.float32),
            pltpu.VMEM((EROWS, LANES), jnp.int32),
            pltpu.VMEM((LANES,), jnp.float32),
            pltpu.VMEM((TMPW + 8,), jnp.float32),
            pltpu.VMEM((G,), jnp.float32),
            pltpu.VMEM_SHARED((CW,), jnp.float32),
            pltpu.VMEM_SHARED((NB,), jnp.float32),
            pltpu.VMEM_SHARED((2 * LANES,), jnp.float32),
        ],
    )


def _sc_build(src2, dst2, batch2):
    return _sc_build_fn(False)(src2, dst2, batch2)


def _tc_body(c_ref, x_ref, fp_ref, rp_ref, w_ref, cnt_ref, out_ref):
    newx = x_ref[...] + fp_ref[...]
    s = jnp.dot(c_ref[...], newx, preferred_element_type=jnp.float32)
    s = s / jnp.maximum(cnt_ref[...], 1.0)
    out_ref[...] = (jnp.dot(s, w_ref[...], preferred_element_type=jnp.float32)
                    * rp_ref[...])


def kernel(x, edge_index, batch, feature_prompt, readout_prompt, W_gnn):
    src = edge_index[0].astype(jnp.int32)
    dst = edge_index[1].astype(jnp.int32)
    b32 = batch.astype(jnp.int32)

    src2 = jnp.pad(src, (0, EPAD - E), constant_values=N).reshape(
        NTILES * EROWS, LANES)
    dst2 = jnp.pad(dst, (0, EPAD - E), constant_values=NB - 1).reshape(
        NTILES * EROWS, LANES)
    batch2 = jnp.pad(b32, (0, NB - N)).reshape(BROWS, LANES)

    c_flat, cnt = _sc_build(src2, dst2, batch2)
    c = c_flat.reshape(G, NPC)

    x_pad = jnp.pad(x, ((0, NPC - N), (0, 0)))
    out = pl.pallas_call(
        _tc_body,
        out_shape=jax.ShapeDtypeStruct((G, D), jnp.float32),
    )(c, x_pad, feature_prompt, readout_prompt, W_gnn, cnt.reshape(G, 1))
    return out
